# Initial kernel scaffold; baseline (speedup 1.0000x reference)
import jax
import jax.numpy as jnp
from jax.experimental import pallas as pl


def kernel(x, neuron_regions, is_left, eid, W, b):
    raise NotImplementedError("write your pallas kernel here")


# speedup vs baseline: 2.1966x; 2.1966x over previous
import jax
import jax.numpy as jnp
from jax.experimental import pallas as pl
from jax.experimental.pallas import tpu as pltpu

_N_CH = 16


def _stitch_body(nr_ref, w_ref, b_ref, x_ref, o_ref):
    n = w_ref.shape[0]
    nr = nr_ref[...]
    m = (nr == 0)
    mf = m.astype(jnp.float32)
    j_iota = jax.lax.broadcasted_iota(jnp.int32, (n, n), 0)
    c_iota = jax.lax.broadcasted_iota(jnp.int32, (n, n), 1)
    tri = jnp.where(j_iota <= c_iota, jnp.float32(1.0), jnp.float32(0.0))
    rank = (jnp.dot(mf, tri, preferred_element_type=jnp.float32) - 1.0
            ).astype(jnp.int32)
    total = jnp.sum(mf).astype(jnp.int32)
    onehot = jnp.where(
        (m & (rank == j_iota)) | ((j_iota >= total) & (c_iota == 0)),
        jnp.float32(1.0), jnp.float32(0.0))
    w_eff = jax.lax.dot_general(
        onehot, w_ref[...], (((0,), (0,)), ((), ())),
        preferred_element_type=jnp.float32)
    o_ref[...] = jnp.dot(x_ref[...], w_eff,
                         preferred_element_type=jnp.float32) + b_ref[...]


def kernel(x, neuron_regions, is_left, eid, W, b):
    Bx, Tx, N = x.shape
    n_ch = W.shape[1]
    rows = Bx * Tx
    x2 = x.reshape(rows, N)
    nr0 = neuron_regions[:1, :]
    b2 = b.reshape(1, n_ch)

    block_rows = 8192
    grid = (rows // block_rows,)
    out = pl.pallas_call(
        _stitch_body,
        grid=grid,
        in_specs=[
            pl.BlockSpec((1, N), lambda i: (0, 0)),
            pl.BlockSpec((N, n_ch), lambda i: (0, 0)),
            pl.BlockSpec((1, n_ch), lambda i: (0, 0)),
            pl.BlockSpec((block_rows, N), lambda i: (i, 0)),
        ],
        out_specs=pl.BlockSpec((block_rows, n_ch), lambda i: (i, 0)),
        out_shape=jax.ShapeDtypeStruct((rows, n_ch), jnp.float32),
        compiler_params=pltpu.CompilerParams(
            dimension_semantics=("arbitrary",)),
    )(nr0, W, b2, x2)
    return out.reshape(Bx, Tx, n_ch)


# speedup vs baseline: 2.2484x; 1.0236x over previous
import jax
import jax.numpy as jnp
from jax.experimental import pallas as pl
from jax.experimental.pallas import tpu as pltpu

_N_CH = 16


def _stitch_body(nr_ref, w_ref, b_ref, x_ref, o_ref, weff_ref):
    @pl.when(pl.program_id(0) == 0)
    def _build_weff():
        n = w_ref.shape[0]
        nr = nr_ref[...]
        m = (nr == 0)
        mf = m.astype(jnp.float32)
        j_iota = jax.lax.broadcasted_iota(jnp.int32, (n, n), 0)
        c_iota = jax.lax.broadcasted_iota(jnp.int32, (n, n), 1)
        tri = jnp.where(j_iota <= c_iota, jnp.float32(1.0), jnp.float32(0.0))
        rank = (jnp.dot(mf, tri, preferred_element_type=jnp.float32) - 1.0
                ).astype(jnp.int32)
        total = jnp.sum(mf).astype(jnp.int32)
        onehot = jnp.where(
            (m & (rank == j_iota)) | ((j_iota >= total) & (c_iota == 0)),
            jnp.float32(1.0), jnp.float32(0.0))
        weff_ref[...] = jax.lax.dot_general(
            onehot, w_ref[...], (((0,), (0,)), ((), ())),
            preferred_element_type=jnp.float32)

    o_ref[...] = jnp.dot(x_ref[...], weff_ref[...],
                         preferred_element_type=jnp.float32) + b_ref[...]


def kernel(x, neuron_regions, is_left, eid, W, b):
    Bx, Tx, N = x.shape
    n_ch = W.shape[1]
    rows = Bx * Tx
    x2 = x.reshape(rows, N)
    nr0 = neuron_regions[:1, :]
    b2 = b.reshape(1, n_ch)

    block_rows = 16384
    grid = (rows // block_rows,)
    out = pl.pallas_call(
        _stitch_body,
        grid=grid,
        in_specs=[
            pl.BlockSpec((1, N), lambda i: (0, 0)),
            pl.BlockSpec((N, n_ch), lambda i: (0, 0)),
            pl.BlockSpec((1, n_ch), lambda i: (0, 0)),
            pl.BlockSpec((block_rows, N), lambda i: (i, 0)),
        ],
        out_specs=pl.BlockSpec((block_rows, n_ch), lambda i: (i, 0)),
        out_shape=jax.ShapeDtypeStruct((rows, n_ch), jnp.float32),
        scratch_shapes=[pltpu.VMEM((N, n_ch), jnp.float32)],
        compiler_params=pltpu.CompilerParams(
            dimension_semantics=("arbitrary",)),
    )(nr0, W, b2, x2)
    return out.reshape(Bx, Tx, n_ch)
